# baseline (speedup 1.0000x reference)
import numpy as np
import jax
import jax.numpy as jnp
from jax.experimental import pallas as pl
from jax.experimental.pallas import tpu as pltpu

_ROWS = 32
_COLS = 1000000
_SUB = 8
_LANES = 500
_CHUNK = _SUB * _LANES
_NCHUNK = _COLS // _CHUNK

_K1 = np.uint32(42)
_KS = (np.uint32(0), np.uint32(42), np.uint32(42 ^ 0x1BD11BDA))
_ROT = ((13, 15, 26, 6), (17, 29, 16, 24))


def _row_kernel(x_ref, o_ref, e_ref):
    i = pl.program_id(0)
    base = (i * _COLS).astype(jnp.uint32) if hasattr(i, "astype") else None
    base = jnp.asarray(i * _COLS, jnp.uint32)
    sub = jax.lax.broadcasted_iota(jnp.uint32, (_SUB, _LANES), 0)
    lane = jax.lax.broadcasted_iota(jnp.uint32, (_SUB, _LANES), 1)
    off = sub * jnp.uint32(_LANES) + lane

    def body1(c, sacc):
        xb = x_ref[0, c]
        n = base + jnp.asarray(c * _CHUNK, jnp.uint32) + off
        x0 = jnp.zeros((_SUB, _LANES), jnp.uint32)
        x1 = n + _K1
        for it in range(5):
            for r in _ROT[it % 2]:
                x0 = x0 + x1
                x1 = (x1 << jnp.uint32(r)) | (x1 >> jnp.uint32(32 - r))
                x1 = x0 ^ x1
            x0 = x0 + _KS[(it + 1) % 3]
            x1 = x1 + _KS[(it + 2) % 3] + jnp.uint32(it + 1)
        bits = x0 ^ x1
        fb = jax.lax.bitcast_convert_type(
            (bits >> jnp.uint32(9)) | jnp.uint32(0x3F800000), jnp.float32)
        u = jnp.maximum(jnp.float32(1e-10),
                        (fb - jnp.float32(1.0)) + jnp.float32(1e-10))
        g = -jnp.log(-jnp.log(u))
        e = jnp.exp(xb + g)
        e_ref[c] = e
        return sacc + e

    sacc = jax.lax.fori_loop(0, _NCHUNK, body1,
                             jnp.zeros((_SUB, _LANES), jnp.float32))
    inv = jnp.float32(1.0) / jnp.sum(sacc)
    o_ref[0] = e_ref[...] * inv


def kernel(logits):
    x = logits.reshape(_ROWS, _NCHUNK, _SUB, _LANES)
    out = pl.pallas_call(
        _row_kernel,
        grid=(_ROWS,),
        in_specs=[pl.BlockSpec((1, _NCHUNK, _SUB, _LANES),
                               lambda i: (i, 0, 0, 0))],
        out_specs=pl.BlockSpec((1, _NCHUNK, _SUB, _LANES),
                               lambda i: (i, 0, 0, 0)),
        out_shape=jax.ShapeDtypeStruct((_ROWS, _NCHUNK, _SUB, _LANES),
                                       jnp.float32),
        scratch_shapes=[pltpu.VMEM((_NCHUNK, _SUB, _LANES), jnp.float32)],
    )(x)
    return out.reshape(_ROWS, _COLS)


# speedup vs baseline: 1.2800x; 1.2800x over previous
import numpy as np
import jax
import jax.numpy as jnp
from jax.experimental import pallas as pl
from jax.experimental.pallas import tpu as pltpu

_ROWS = 32
_COLS = 1000000
_SUB = 8
_LANES = 1000
_CHUNK = _SUB * _LANES
_NCHUNK = _COLS // _CHUNK

_K1 = np.uint32(42)
_KS = (np.uint32(0), np.uint32(42), np.uint32(42 ^ 0x1BD11BDA))
_ROT = ((13, 15, 26, 6), (17, 29, 16, 24))


def _row_kernel(x_ref, o_ref, e_ref):
    i = pl.program_id(0)
    base = (i * _COLS).astype(jnp.uint32) if hasattr(i, "astype") else None
    base = jnp.asarray(i * _COLS, jnp.uint32)
    sub = jax.lax.broadcasted_iota(jnp.uint32, (_SUB, _LANES), 0)
    lane = jax.lax.broadcasted_iota(jnp.uint32, (_SUB, _LANES), 1)
    off = sub * jnp.uint32(_LANES) + lane

    def body1(c, sacc):
        xb = x_ref[0, c]
        n = base + jnp.asarray(c * _CHUNK, jnp.uint32) + off
        x0 = jnp.zeros((_SUB, _LANES), jnp.uint32)
        x1 = n + _K1
        for it in range(5):
            for r in _ROT[it % 2]:
                x0 = x0 + x1
                x1 = (x1 << jnp.uint32(r)) | (x1 >> jnp.uint32(32 - r))
                x1 = x0 ^ x1
            x0 = x0 + _KS[(it + 1) % 3]
            x1 = x1 + _KS[(it + 2) % 3] + jnp.uint32(it + 1)
        bits = x0 ^ x1
        fb = jax.lax.bitcast_convert_type(
            (bits >> jnp.uint32(9)) | jnp.uint32(0x3F800000), jnp.float32)
        u = jnp.maximum(jnp.float32(1e-10),
                        (fb - jnp.float32(1.0)) + jnp.float32(1e-10))
        g = -jnp.log(-jnp.log(u))
        e = jnp.exp(xb + g)
        e_ref[c] = e
        return sacc + e

    sacc = jax.lax.fori_loop(0, _NCHUNK, body1,
                             jnp.zeros((_SUB, _LANES), jnp.float32))
    inv = jnp.float32(1.0) / jnp.sum(sacc)
    o_ref[0] = e_ref[...] * inv


def kernel(logits):
    x = logits.reshape(_ROWS, _NCHUNK, _SUB, _LANES)
    out = pl.pallas_call(
        _row_kernel,
        grid=(_ROWS,),
        in_specs=[pl.BlockSpec((1, _NCHUNK, _SUB, _LANES),
                               lambda i: (i, 0, 0, 0))],
        out_specs=pl.BlockSpec((1, _NCHUNK, _SUB, _LANES),
                               lambda i: (i, 0, 0, 0)),
        out_shape=jax.ShapeDtypeStruct((_ROWS, _NCHUNK, _SUB, _LANES),
                                       jnp.float32),
        scratch_shapes=[pltpu.VMEM((_NCHUNK, _SUB, _LANES), jnp.float32)],
    )(x)
    return out.reshape(_ROWS, _COLS)


# speedup vs baseline: 1.3800x; 1.0781x over previous
import numpy as np
import jax
import jax.numpy as jnp
from jax.experimental import pallas as pl
from jax.experimental.pallas import tpu as pltpu

_ROWS = 32
_COLS = 1000000
_SUB = 8
_LANES = 1000
_CHUNK = _SUB * _LANES
_NCHUNK = _COLS // _CHUNK

_K1 = np.uint32(42)
_KS = (np.uint32(0), np.uint32(42), np.uint32(42 ^ 0x1BD11BDA))
_ROT = ((13, 15, 26, 6), (17, 29, 16, 24))


def _row_kernel(x_ref, o_ref, e_ref):
    i = pl.program_id(0)
    base = (i * _COLS).astype(jnp.uint32) if hasattr(i, "astype") else None
    base = jnp.asarray(i * _COLS, jnp.uint32)
    sub = jax.lax.broadcasted_iota(jnp.uint32, (_SUB, _LANES), 0)
    lane = jax.lax.broadcasted_iota(jnp.uint32, (_SUB, _LANES), 1)
    off = sub * jnp.uint32(_LANES) + lane

    def body1(c, sacc):
        acc = sacc
        for k in range(2):
            cc = 2 * c + k
            xb = x_ref[0, cc]
            n = base + jnp.asarray(cc * _CHUNK, jnp.uint32) + off
            x0 = jnp.zeros((_SUB, _LANES), jnp.uint32)
            x1 = n + _K1
            for it in range(5):
                for r in _ROT[it % 2]:
                    x0 = x0 + x1
                    x1 = (x1 << jnp.uint32(r)) | (x1 >> jnp.uint32(32 - r))
                    x1 = x0 ^ x1
                x0 = x0 + _KS[(it + 1) % 3]
                x1 = x1 + _KS[(it + 2) % 3] + jnp.uint32(it + 1)
            bits = x0 ^ x1
            fb = jax.lax.bitcast_convert_type(
                (bits >> jnp.uint32(9)) | jnp.uint32(0x3F800000), jnp.float32)
            u = (fb - jnp.float32(1.0)) + jnp.float32(1e-10)
            e = jnp.exp(xb) / (-jnp.log(u))
            e_ref[cc] = e
            acc = acc + e
        return acc

    sacc = jax.lax.fori_loop(0, _NCHUNK // 2, body1,
                             jnp.zeros((_SUB, _LANES), jnp.float32))
    inv = jnp.float32(1.0) / jnp.sum(sacc)
    o_ref[0] = e_ref[...] * inv


def kernel(logits):
    x = logits.reshape(_ROWS, _NCHUNK, _SUB, _LANES)
    out = pl.pallas_call(
        _row_kernel,
        grid=(_ROWS,),
        in_specs=[pl.BlockSpec((1, _NCHUNK, _SUB, _LANES),
                               lambda i: (i, 0, 0, 0))],
        out_specs=pl.BlockSpec((1, _NCHUNK, _SUB, _LANES),
                               lambda i: (i, 0, 0, 0)),
        out_shape=jax.ShapeDtypeStruct((_ROWS, _NCHUNK, _SUB, _LANES),
                                       jnp.float32),
        scratch_shapes=[pltpu.VMEM((_NCHUNK, _SUB, _LANES), jnp.float32)],
    )(x)
    return out.reshape(_ROWS, _COLS)


# speedup vs baseline: 1.3874x; 1.0054x over previous
import numpy as np
import jax
import jax.numpy as jnp
from jax.experimental import pallas as pl
from jax.experimental.pallas import tpu as pltpu

_ROWS = 32
_COLS = 1000000
_SUB = 8
_LANES = 1000
_CHUNK = _SUB * _LANES
_NCHUNK = _COLS // _CHUNK

_K1 = np.uint32(42)
_KS = (np.uint32(0), np.uint32(42), np.uint32(42 ^ 0x1BD11BDA))
_ROT = ((13, 15, 26, 6), (17, 29, 16, 24))


def _row_kernel(x_ref, o_ref, e_ref):
    i = pl.program_id(0)
    base = (i * _COLS).astype(jnp.uint32) if hasattr(i, "astype") else None
    base = jnp.asarray(i * _COLS, jnp.uint32)
    sub = jax.lax.broadcasted_iota(jnp.uint32, (_SUB, _LANES), 0)
    lane = jax.lax.broadcasted_iota(jnp.uint32, (_SUB, _LANES), 1)
    off = sub * jnp.uint32(_LANES) + lane

    def chunk_e(cc):
        xb = x_ref[0, cc]
        n = base + jnp.asarray(cc * _CHUNK, jnp.uint32) + off
        x0 = jnp.zeros((_SUB, _LANES), jnp.uint32)
        x1 = n + _K1
        for it in range(5):
            for r in _ROT[it % 2]:
                x0 = x0 + x1
                x1 = (x1 << jnp.uint32(r)) | (x1 >> jnp.uint32(32 - r))
                x1 = x0 ^ x1
            x0 = x0 + _KS[(it + 1) % 3]
            x1 = x1 + _KS[(it + 2) % 3] + jnp.uint32(it + 1)
        bits = x0 ^ x1
        fb = jax.lax.bitcast_convert_type(
            (bits >> jnp.uint32(9)) | jnp.uint32(0x3F800000), jnp.float32)
        u = (fb - jnp.float32(1.0)) + jnp.float32(1e-10)
        e = jnp.exp(xb) / (-jnp.log(u))
        e_ref[cc] = e
        return e

    def body1(c, sacc):
        acc = sacc + chunk_e(2 * c)
        return acc + chunk_e(2 * c + 1)

    sacc = jax.lax.fori_loop(0, _NCHUNK // 2, body1,
                             jnp.zeros((_SUB, _LANES), jnp.float32))
    if _NCHUNK % 2:
        sacc = sacc + chunk_e(_NCHUNK - 1)
    inv = jnp.float32(1.0) / jnp.sum(sacc)
    o_ref[0] = e_ref[...] * inv


def kernel(logits):
    x = logits.reshape(_ROWS, _NCHUNK, _SUB, _LANES)
    out = pl.pallas_call(
        _row_kernel,
        grid=(_ROWS,),
        in_specs=[pl.BlockSpec((1, _NCHUNK, _SUB, _LANES),
                               lambda i: (i, 0, 0, 0))],
        out_specs=pl.BlockSpec((1, _NCHUNK, _SUB, _LANES),
                               lambda i: (i, 0, 0, 0)),
        out_shape=jax.ShapeDtypeStruct((_ROWS, _NCHUNK, _SUB, _LANES),
                                       jnp.float32),
        scratch_shapes=[pltpu.VMEM((_NCHUNK, _SUB, _LANES), jnp.float32)],
    )(x)
    return out.reshape(_ROWS, _COLS)
